# Initial kernel scaffold; baseline (speedup 1.0000x reference)
import jax
import jax.numpy as jnp
from jax.experimental import pallas as pl


def kernel(neigh_0, neigh_1, neigh_2, neigh_3, neigh_4, neigh_5, x, W0, b0, g0, e0, W1, b1, g1, e1, W2, b2, g2, e2, W3, b3, g3, e3, W4, b4, g4, e4, W5, b5, g5, e5, W6, b6, g6, e6, W7, b7, g7, e7, W8, b8, g8, e8, W9, b9, g9, e9, W10, b10, g10, e10, W11, b11, g11, e11, W12, b12, g12, e12, Wfc, bfc):
    raise NotImplementedError("write your pallas kernel here")


# speedup vs baseline: 1.6929x; 1.6929x over previous
import functools

import jax
import jax.numpy as jnp
from jax import lax
from jax.experimental import pallas as pl
from jax.experimental.pallas import tpu as pltpu
from jax.experimental.pallas import tpu_sc as plsc

_CHS = [3, 32, 64, 128, 256, 512, 1024]
_NS = [40962, 10242, 2562, 642, 162, 42]
_NPS = [41216, 10496, 2816, 704, 192, 48]
_BRS = [256, 256, 256, 64, 64, 48]
_NW = 32


def _gather_tile_rows(d):
    return 64 if d >= 1024 else 128


def _sc_gather(table, idx):
    v, d = table.shape
    b = idx.shape[0]
    t = min(_gather_tile_rows(d), b)
    nt = -(-b // t)
    ntw = -(-nt // _NW)
    mesh = plsc.VectorSubcoreMesh(core_axis_name="c", subcore_axis_name="s")

    @functools.partial(
        pl.kernel,
        out_type=jax.ShapeDtypeStruct((b, d), jnp.float32),
        mesh=mesh,
        scratch_types=[
            pltpu.VMEM((t,), jnp.int32),
            pltpu.VMEM((t, d), jnp.float32),
            pltpu.SemaphoreType.DMA,
        ],
        compiler_params=pltpu.CompilerParams(use_tc_tiling_on_sc=False),
    )
    def gk(table_hbm, idx_hbm, out_hbm, idx_v, rows_v, sem):
        wid = lax.axis_index("s") * 2 + lax.axis_index("c")

        def body(i, carry):
            tile = wid + i * _NW

            @pl.when(tile < nt)
            def _():
                base = jnp.minimum(tile * t, b - t)
                pltpu.sync_copy(idx_hbm.at[pl.ds(base, t)], idx_v)
                pltpu.async_copy(table_hbm.at[idx_v], rows_v, sem).wait()
                pltpu.sync_copy(rows_v, out_hbm.at[pl.ds(base, t)])

            return carry

        lax.fori_loop(0, ntw, body, 0)

    return gk(table, idx)


def _tc_conv(x, mat6, w_id, w_nb, bge, n, br):
    n_p, cin_p = x.shape
    c6 = mat6.shape[1]
    cout = w_id.shape[1]
    nb = n_p // br

    def body(x_ref, m6_ref, wid_ref, wnb_ref, bge_ref, out_ref, yacc, stats):
        p = pl.program_id(0)
        j = pl.program_id(1)

        @pl.when(p == 0)
        def _():
            y = (
                jnp.dot(x_ref[...], wid_ref[...], preferred_element_type=jnp.float32)
                + jnp.dot(m6_ref[...], wnb_ref[...], preferred_element_type=jnp.float32)
                + bge_ref[0:1, :]
            )
            yacc[pl.ds(j * br, br), :] = y
            rows = j * br + lax.broadcasted_iota(jnp.int32, (br, 1), 0)
            ym = jnp.where(rows < n, y, 0.0)
            s1 = jnp.sum(ym, axis=0, keepdims=True)
            s2 = jnp.sum(ym * ym, axis=0, keepdims=True)

            @pl.when(j == 0)
            def _():
                stats[0:1, :] = s1
                stats[1:2, :] = s2

            @pl.when(j > 0)
            def _():
                stats[0:1, :] += s1
                stats[1:2, :] += s2

        @pl.when(p == 1)
        def _():
            m = stats[0:1, :] * (1.0 / n)
            var = stats[1:2, :] * (1.0 / n) - m * m
            scale = bge_ref[1:2, :] * lax.rsqrt(var + 1e-5)
            shift = bge_ref[2:3, :] - m * scale
            yv = yacc[pl.ds(j * br, br), :] * scale + shift
            out_ref[...] = jnp.where(yv >= 0, yv, 0.1 * yv)

    return pl.pallas_call(
        body,
        grid=(2, nb),
        in_specs=[
            pl.BlockSpec((br, cin_p), lambda p, j: ((1 - p) * j, 0)),
            pl.BlockSpec((br, c6), lambda p, j: ((1 - p) * j, 0)),
            pl.BlockSpec(w_id.shape, lambda p, j: (0, 0)),
            pl.BlockSpec(w_nb.shape, lambda p, j: (0, 0)),
            pl.BlockSpec((3, cout), lambda p, j: (0, 0)),
        ],
        out_specs=pl.BlockSpec((br, cout), lambda p, j: (p * j, 0)),
        out_shape=jax.ShapeDtypeStruct((n_p, cout), jnp.float32),
        scratch_shapes=[
            pltpu.VMEM((n_p, cout), jnp.float32),
            pltpu.VMEM((2, cout), jnp.float32),
        ],
    )(x, mat6, w_id, w_nb, bge)


def _tc_poolmean(g7, br):
    _, nc_p, c = g7.shape
    nb = nc_p // br

    def body(g_ref, out_ref):
        out_ref[...] = jnp.sum(g_ref[...], axis=0) * (1.0 / 7.0)

    return pl.pallas_call(
        body,
        grid=(nb,),
        in_specs=[pl.BlockSpec((7, br, c), lambda j: (0, j, 0))],
        out_specs=pl.BlockSpec((br, c), lambda j: (j, 0)),
        out_shape=jax.ShapeDtypeStruct((nc_p, c), jnp.float32),
    )(g7)


def _tc_final(x5, wfc, bfc, n):
    n_p, c = x5.shape

    def body(x_ref, w_ref, b_ref, out_ref):
        rows = lax.broadcasted_iota(jnp.int32, (n_p, 1), 0)
        xm = jnp.where(rows < n, x_ref[...], 0.0)
        s = jnp.sum(xm, axis=0, keepdims=True) * (1.0 / n)
        out_ref[...] = (
            jnp.dot(s, w_ref[...], preferred_element_type=jnp.float32) + b_ref[0:1, :]
        )

    return pl.pallas_call(
        body,
        out_shape=jax.ShapeDtypeStruct((1, 2), jnp.float32),
    )(x5, wfc, bfc.reshape(1, 2))


def _conv_idx(no2, n, n_p):
    idx6 = jnp.pad(no2[:, 1:7], ((0, n_p - n), (0, 0)))
    return idx6.reshape(-1)


def _pool_idx(no_fine_flat, nc, nc_p):
    idx = no_fine_flat[: nc * 7].reshape(nc, 7)
    idx = jnp.pad(idx, ((0, nc_p - nc), (0, 0)))
    return idx.T.reshape(-1)


def kernel(neigh_0, neigh_1, neigh_2, neigh_3, neigh_4, neigh_5, x,
           W0, b0, g0, e0, W1, b1, g1, e1, W2, b2, g2, e2, W3, b3, g3, e3,
           W4, b4, g4, e4, W5, b5, g5, e5, W6, b6, g6, e6, W7, b7, g7, e7,
           W8, b8, g8, e8, W9, b9, g9, e9, W10, b10, g10, e10,
           W11, b11, g11, e11, W12, b12, g12, e12, Wfc, bfc):
    neighs = (neigh_0, neigh_1, neigh_2, neigh_3, neigh_4, neigh_5)
    ws = (W0, W1, W2, W3, W4, W5, W6, W7, W8, W9, W10, W11, W12)
    bges = (
        (b0, g0, e0), (b1, g1, e1), (b2, g2, e2), (b3, g3, e3),
        (b4, g4, e4), (b5, g5, e5), (b6, g6, e6), (b7, g7, e7),
        (b8, g8, e8), (b9, g9, e9), (b10, g10, e10), (b11, g11, e11),
        (b12, g12, e12),
    )

    def conv(h, idx6, ci, cin, level):
        n, n_p, br = _NS[level], _NPS[level], _BRS[level]
        w = ws[ci]
        cin_p = h.shape[1]
        if cin_p == cin:
            w_id = w[:cin]
        else:
            w_id = jnp.pad(w[:cin], ((0, cin_p - cin), (0, 0)))
        w_nb = w[cin:].reshape(6, cin, -1)
        cout = w_nb.shape[-1]
        if cin_p != cin:
            w_nb = jnp.pad(w_nb, ((0, 0), (0, cin_p - cin), (0, 0)))
        w_nb = w_nb.reshape(6 * cin_p, cout)
        b, g, e = bges[ci]
        bge = jnp.stack([b, g, e])
        mat6 = _sc_gather(h, idx6).reshape(n_p, 6 * cin_p)
        return _tc_conv(h, mat6, w_id, w_nb, bge, n, br)

    n0, np0 = _NS[0], _NPS[0]
    h = jnp.pad(x, ((0, np0 - n0), (0, 16 - _CHS[0])))
    no0 = neigh_0.reshape(_NS[0], 7)
    idx6_0 = _conv_idx(no0, n0, np0)
    h = conv(h, idx6_0, 0, _CHS[0], 0)
    h = conv(h, idx6_0, 1, _CHS[1], 0)
    h = conv(h, idx6_0, 2, _CHS[1], 0)

    ci = 3
    for l in range(1, 6):
        nc, nc_p, br = _NS[l], _NPS[l], _BRS[l]
        c = _CHS[l]
        idxp = _pool_idx(neighs[l - 1], nc, nc_p)
        g7 = _sc_gather(h, idxp).reshape(7, nc_p, c)
        h = _tc_poolmean(g7, br)
        no_l = neighs[l].reshape(nc, 7)
        idx6_l = _conv_idx(no_l, nc, nc_p)
        h = conv(h, idx6_l, ci, _CHS[l], l)
        h = conv(h, idx6_l, ci + 1, _CHS[l + 1], l)
        ci += 2

    return _tc_final(h, Wfc, bfc, _NS[5])


# speedup vs baseline: 1.8541x; 1.0952x over previous
import functools

import jax
import jax.numpy as jnp
from jax import lax
from jax.experimental import pallas as pl
from jax.experimental.pallas import tpu as pltpu
from jax.experimental.pallas import tpu_sc as plsc

_CHS = [3, 32, 64, 128, 256, 512, 1024]
_NS = [40962, 10242, 2562, 642, 162, 42]
_NPS = [41216, 10496, 2816, 704, 192, 48]
_BRS = [256, 256, 256, 64, 64, 48]
_NW = 32


def _gather_plan(b, d):
    t_max = max(8, (230 * 1024 // (4 * (d + 1))) // 8 * 8)
    k = -(-b // (_NW * t_max))
    t = -(-(-(-b // (_NW * k))) // 8) * 8
    return t, k


def _sc_gather(table, idx):
    v, d = table.shape
    b = idx.shape[0]
    t, k = _gather_plan(b, d)
    nt = -(-b // t)
    mesh = plsc.VectorSubcoreMesh(core_axis_name="c", subcore_axis_name="s")
    nslots = min(2, k)

    @functools.partial(
        pl.kernel,
        out_type=jax.ShapeDtypeStruct((b, d), jnp.float32),
        mesh=mesh,
        scratch_types=[
            [pltpu.VMEM((t,), jnp.int32) for _ in range(nslots)],
            [pltpu.VMEM((t, d), jnp.float32) for _ in range(nslots)],
            [pltpu.SemaphoreType.DMA for _ in range(3 * nslots)],
        ],
        compiler_params=pltpu.CompilerParams(use_tc_tiling_on_sc=False),
    )
    def gk(table_hbm, idx_hbm, out_hbm, idx_v, rows_v, sems):
        wid = lax.axis_index("s") * 2 + lax.axis_index("c")

        def base(i):
            tile = jnp.minimum(wid + i * _NW, nt - 1)
            return jnp.minimum(tile * t, b - t)

        idx_h = [None] * k
        g_h = [None] * k
        w_h = [None] * k
        for i in range(nslots):
            idx_h[i] = pltpu.async_copy(
                idx_hbm.at[pl.ds(base(i), t)], idx_v[i], sems[i]
            )
        for i in range(k):
            s = i % 2
            idx_h[i].wait()
            if i >= 2:
                w_h[i - 2].wait()
            g_h[i] = pltpu.async_copy(
                table_hbm.at[idx_v[s]], rows_v[s], sems[nslots + s]
            )
            if i >= 1:
                g_h[i - 1].wait()
                w_h[i - 1] = pltpu.async_copy(
                    rows_v[1 - s], out_hbm.at[pl.ds(base(i - 1), t)],
                    sems[2 * nslots + (1 - s)],
                )
                if 2 <= i + 1 < k:
                    idx_h[i + 1] = pltpu.async_copy(
                        idx_hbm.at[pl.ds(base(i + 1), t)], idx_v[1 - s],
                        sems[1 - s],
                    )
        g_h[k - 1].wait()
        w_h[k - 1] = pltpu.async_copy(
            rows_v[(k - 1) % 2], out_hbm.at[pl.ds(base(k - 1), t)],
            sems[2 * nslots + ((k - 1) % 2)],
        )
        if k >= 2:
            w_h[k - 2].wait()
        w_h[k - 1].wait()

    return gk(table, idx)


def _tc_conv(x, mat6, w_id, w_nb, bge, n, br):
    n_p, cin_p = x.shape
    c6 = mat6.shape[1]
    cout = w_id.shape[1]
    nb = n_p // br

    def body(x_ref, m6_ref, wid_ref, wnb_ref, bge_ref, out_ref, yacc, stats):
        p = pl.program_id(0)
        j = pl.program_id(1)

        @pl.when(p == 0)
        def _():
            y = (
                jnp.dot(x_ref[...], wid_ref[...], preferred_element_type=jnp.float32)
                + jnp.dot(m6_ref[...], wnb_ref[...], preferred_element_type=jnp.float32)
                + bge_ref[0:1, :]
            )
            yacc[pl.ds(j * br, br), :] = y
            rows = j * br + lax.broadcasted_iota(jnp.int32, (br, 1), 0)
            ym = jnp.where(rows < n, y, 0.0)
            s1 = jnp.sum(ym, axis=0, keepdims=True)
            s2 = jnp.sum(ym * ym, axis=0, keepdims=True)

            @pl.when(j == 0)
            def _():
                stats[0:1, :] = s1
                stats[1:2, :] = s2

            @pl.when(j > 0)
            def _():
                stats[0:1, :] += s1
                stats[1:2, :] += s2

        @pl.when(p == 1)
        def _():
            m = stats[0:1, :] * (1.0 / n)
            var = stats[1:2, :] * (1.0 / n) - m * m
            scale = bge_ref[1:2, :] * lax.rsqrt(var + 1e-5)
            shift = bge_ref[2:3, :] - m * scale
            yv = yacc[pl.ds(j * br, br), :] * scale + shift
            out_ref[...] = jnp.where(yv >= 0, yv, 0.1 * yv)

    return pl.pallas_call(
        body,
        grid=(2, nb),
        in_specs=[
            pl.BlockSpec((br, cin_p), lambda p, j: ((1 - p) * j, 0)),
            pl.BlockSpec((br, c6), lambda p, j: ((1 - p) * j, 0)),
            pl.BlockSpec(w_id.shape, lambda p, j: (0, 0)),
            pl.BlockSpec(w_nb.shape, lambda p, j: (0, 0)),
            pl.BlockSpec((3, cout), lambda p, j: (0, 0)),
        ],
        out_specs=pl.BlockSpec((br, cout), lambda p, j: (p * j, 0)),
        out_shape=jax.ShapeDtypeStruct((n_p, cout), jnp.float32),
        scratch_shapes=[
            pltpu.VMEM((n_p, cout), jnp.float32),
            pltpu.VMEM((2, cout), jnp.float32),
        ],
    )(x, mat6, w_id, w_nb, bge)


def _tc_poolmean(g7, br):
    _, nc_p, c = g7.shape
    nb = nc_p // br

    def body(g_ref, out_ref):
        out_ref[...] = jnp.sum(g_ref[...], axis=0) * (1.0 / 7.0)

    return pl.pallas_call(
        body,
        grid=(nb,),
        in_specs=[pl.BlockSpec((7, br, c), lambda j: (0, j, 0))],
        out_specs=pl.BlockSpec((br, c), lambda j: (j, 0)),
        out_shape=jax.ShapeDtypeStruct((nc_p, c), jnp.float32),
    )(g7)


def _tc_final(x5, wfc, bfc, n):
    n_p, c = x5.shape

    def body(x_ref, w_ref, b_ref, out_ref):
        rows = lax.broadcasted_iota(jnp.int32, (n_p, 1), 0)
        xm = jnp.where(rows < n, x_ref[...], 0.0)
        s = jnp.sum(xm, axis=0, keepdims=True) * (1.0 / n)
        out_ref[...] = (
            jnp.dot(s, w_ref[...], preferred_element_type=jnp.float32) + b_ref[0:1, :]
        )

    return pl.pallas_call(
        body,
        out_shape=jax.ShapeDtypeStruct((1, 2), jnp.float32),
    )(x5, wfc, bfc.reshape(1, 2))


def _conv_idx(no2, n, n_p):
    idx6 = jnp.pad(no2[:, 1:7], ((0, n_p - n), (0, 0)))
    return idx6.reshape(-1)


def _pool_idx(no_fine_flat, nc, nc_p):
    idx = no_fine_flat[: nc * 7].reshape(nc, 7)
    idx = jnp.pad(idx, ((0, nc_p - nc), (0, 0)))
    return idx.T.reshape(-1)


def kernel(neigh_0, neigh_1, neigh_2, neigh_3, neigh_4, neigh_5, x,
           W0, b0, g0, e0, W1, b1, g1, e1, W2, b2, g2, e2, W3, b3, g3, e3,
           W4, b4, g4, e4, W5, b5, g5, e5, W6, b6, g6, e6, W7, b7, g7, e7,
           W8, b8, g8, e8, W9, b9, g9, e9, W10, b10, g10, e10,
           W11, b11, g11, e11, W12, b12, g12, e12, Wfc, bfc):
    neighs = (neigh_0, neigh_1, neigh_2, neigh_3, neigh_4, neigh_5)
    ws = (W0, W1, W2, W3, W4, W5, W6, W7, W8, W9, W10, W11, W12)
    bges = (
        (b0, g0, e0), (b1, g1, e1), (b2, g2, e2), (b3, g3, e3),
        (b4, g4, e4), (b5, g5, e5), (b6, g6, e6), (b7, g7, e7),
        (b8, g8, e8), (b9, g9, e9), (b10, g10, e10), (b11, g11, e11),
        (b12, g12, e12),
    )

    def conv(h, idx6, ci, cin, level):
        n, n_p, br = _NS[level], _NPS[level], _BRS[level]
        w = ws[ci]
        cin_p = h.shape[1]
        if cin_p == cin:
            w_id = w[:cin]
        else:
            w_id = jnp.pad(w[:cin], ((0, cin_p - cin), (0, 0)))
        w_nb = w[cin:].reshape(6, cin, -1)
        cout = w_nb.shape[-1]
        if cin_p != cin:
            w_nb = jnp.pad(w_nb, ((0, 0), (0, cin_p - cin), (0, 0)))
        w_nb = w_nb.reshape(6 * cin_p, cout)
        b, g, e = bges[ci]
        bge = jnp.stack([b, g, e])
        mat6 = _sc_gather(h, idx6).reshape(n_p, 6 * cin_p)
        return _tc_conv(h, mat6, w_id, w_nb, bge, n, br)

    n0, np0 = _NS[0], _NPS[0]
    h = jnp.pad(x, ((0, np0 - n0), (0, 16 - _CHS[0])))
    no0 = neigh_0.reshape(_NS[0], 7)
    idx6_0 = _conv_idx(no0, n0, np0)
    h = conv(h, idx6_0, 0, _CHS[0], 0)
    h = conv(h, idx6_0, 1, _CHS[1], 0)
    h = conv(h, idx6_0, 2, _CHS[1], 0)

    ci = 3
    for l in range(1, 6):
        nc, nc_p, br = _NS[l], _NPS[l], _BRS[l]
        c = _CHS[l]
        idxp = _pool_idx(neighs[l - 1], nc, nc_p)
        g7 = _sc_gather(h, idxp).reshape(7, nc_p, c)
        h = _tc_poolmean(g7, br)
        no_l = neighs[l].reshape(nc, 7)
        idx6_l = _conv_idx(no_l, nc, nc_p)
        h = conv(h, idx6_l, ci, _CHS[l], l)
        h = conv(h, idx6_l, ci + 1, _CHS[l + 1], l)
        ci += 2

    return _tc_final(h, Wfc, bfc, _NS[5])
